# baseline (speedup 1.0000x reference)
import functools

import numpy as np
import jax
import jax.numpy as jnp
from jax import lax
from jax.experimental import pallas as pl
from jax.experimental.pallas import tpu as pltpu
from jax.experimental.pallas import tpu_sc as plsc

NF = 26
VOCAB = 1000
TOTAL = NF * VOCAB
EMB = 16
B = 1024
ND = 13
NPOS = NF * NF
DNN_IN = NPOS * EMB
HALF = (NPOS // 2) * EMB
NPAIRS = (NF * (NF - 1)) // 2

_pairs = [(i, j) for i in range(NF) for j in range(i + 1, NF)]
_left = _pairs + [(d, d) for d in range(13)]
_right = [(j, i) for (i, j) in _pairs] + [(d, d) for d in range(13, NF)]
_order = _left + _right
_PI = np.array([p[0] for p in _order], dtype=np.int32)
_PJ = np.array([p[1] for p in _order], dtype=np.int32)
_OLD = _PI * NF + _PJ
_COLBASE = _PJ * VOCAB * NF + _PI

NW = 32
NROWS = B * NPOS
RPW = NROWS // NW
CH = 128
NCH = RPW // CH
GRP = 13
NGRP = NCH // GRP
FROWS = B * NF
FRPW = FROWS // NW
FCH = 64
FNCH = FRPW // FCH


def _sc_gather(emb_flat, idx3, f16, idxf3):
    mesh = plsc.VectorSubcoreMesh(core_axis_name="c", subcore_axis_name="s")
    nc = mesh.num_cores

    @functools.partial(
        pl.kernel,
        out_type=[
            jax.ShapeDtypeStruct((NROWS, EMB), jnp.float32),
            jax.ShapeDtypeStruct((FROWS, EMB), jnp.float32),
        ],
        mesh=mesh,
        compiler_params=pltpu.CompilerParams(use_tc_tiling_on_sc=False),
        scratch_types=(
            [pltpu.VMEM((NCH, CH), jnp.int32),
             pltpu.VMEM((FNCH, FCH), jnp.int32)]
            + [pltpu.VMEM((CH, EMB), jnp.float32) for _ in range(GRP)]
            + [pltpu.VMEM((FCH, EMB), jnp.float32) for _ in range(FNCH)]
            + [pltpu.SemaphoreType.DMA, pltpu.SemaphoreType.DMA]
        ),
    )
    def k(emb_hbm, idx_hbm, f_hbm, idxf_hbm, gout, fout, idx_v, idxf_v, *rest):
        bufs = rest[:GRP]
        fbufs = rest[GRP:GRP + FNCH]
        sem_g = rest[GRP + FNCH]
        sem_o = rest[GRP + FNCH + 1]
        wid = lax.axis_index("s") * nc + lax.axis_index("c")
        pltpu.sync_copy(idx_hbm.at[wid], idx_v)
        pltpu.sync_copy(idxf_hbm.at[wid], idxf_v)

        fbase = wid * FRPW
        fdescs = [pltpu.async_copy(f_hbm.at[idxf_v.at[c]], fbufs[c], sem_g)
                  for c in range(FNCH)]
        for d in fdescs:
            d.wait()
        odescs = [pltpu.async_copy(
            fbufs[c], fout.at[pl.ds(fbase + c * FCH, FCH)], sem_o)
            for c in range(FNCH)]
        for d in odescs:
            d.wait()

        base = wid * RPW

        def grp_body(g, carry):
            off = g * GRP
            descs = [pltpu.async_copy(
                emb_hbm.at[idx_v.at[off + c]], bufs[c], sem_g)
                for c in range(GRP)]
            for d in descs:
                d.wait()
            outs = [pltpu.async_copy(
                bufs[c], gout.at[pl.ds(base + (off + c) * CH, CH)], sem_o)
                for c in range(GRP)]
            for d in outs:
                d.wait()
            return carry

        lax.fori_loop(0, NGRP, grp_body, 0)

    return k(emb_flat, idx3, f16, idxf3)


def _tc1_body(g_ref, xd_ref, wd_ref, bd_ref, w1_ref, b1_ref, out_ref, fm2_ref):
    gt = g_ref[...]
    dense = lax.dot_general(xd_ref[...], wd_ref[...],
                            (((1,), (1,)), ((), ())),
                            preferred_element_type=jnp.float32)
    dense = jnp.maximum(dense + bd_ref[...][None, :], 0.0)
    z = gt + dense
    out_ref[...] = (lax.dot_general(z, w1_ref[...],
                                    (((1,), (1,)), ((), ())),
                                    preferred_element_type=jnp.float32)
                    + b1_ref[...][None, :])
    lhs = gt[:, :HALF]
    rhs = gt[:, HALF:]
    mask = lax.broadcasted_iota(jnp.int32, (128, HALF), 1) < NPAIRS * EMB
    prod = jnp.where(mask, lhs * rhs, 0.0)
    fm2 = jnp.sum(prod, axis=1, keepdims=True)
    fm2_ref[...] = jnp.broadcast_to(fm2, (128, 128))


def _tc1(g, xd, wd, bd, w1, b1):
    return pl.pallas_call(
        _tc1_body,
        grid=(B // 128,),
        in_specs=[
            pl.BlockSpec((128, DNN_IN), lambda b: (b, 0)),
            pl.BlockSpec((128, ND), lambda b: (b, 0)),
            pl.BlockSpec((DNN_IN, ND), lambda b: (0, 0)),
            pl.BlockSpec((DNN_IN,), lambda b: (0,)),
            pl.BlockSpec((256, DNN_IN), lambda b: (0, 0)),
            pl.BlockSpec((256,), lambda b: (0,)),
        ],
        out_specs=[
            pl.BlockSpec((128, 256), lambda b: (b, 0)),
            pl.BlockSpec((128, 128), lambda b: (b, 0)),
        ],
        out_shape=[
            jax.ShapeDtypeStruct((B, 256), jnp.float32),
            jax.ShapeDtypeStruct((B, 128), jnp.float32),
        ],
    )(g, xd, wd, bd, w1, b1)


def _tc2_body(x_ref, fm2_ref, fg_ref, xd_ref, wfm_ref, bfm_ref, bias_ref,
              g1_ref, be1_ref, w2t_ref, b2_ref, g2_ref, be2_ref,
              wo_ref, bo_ref, out_ref):
    eps = 1e-5
    x = x_ref[...]
    m1 = jnp.mean(x, axis=0)
    v1 = jnp.mean(x * x, axis=0) - m1 * m1
    h1 = (x - m1[None, :]) * lax.rsqrt(v1[None, :] + eps)
    h1 = jnp.maximum(h1 * g1_ref[...][None, :] + be1_ref[...][None, :], 0.0)
    h2 = lax.dot_general(h1, w2t_ref[...], (((1,), (1,)), ((), ())),
                         preferred_element_type=jnp.float32)
    h2 = h2 + b2_ref[...][None, :]
    m2 = jnp.mean(h2, axis=0)
    v2 = jnp.mean(h2 * h2, axis=0) - m2 * m2
    h2 = (h2 - m2[None, :]) * lax.rsqrt(v2[None, :] + eps)
    h2 = jnp.maximum(h2 * g2_ref[...][None, :] + be2_ref[...][None, :], 0.0)
    d = jnp.sum(h2 * wo_ref[...], axis=1, keepdims=True) + bo_ref[...][None, :]
    fm1 = (jnp.sum(fg_ref[...], axis=1, keepdims=True)
           + bias_ref[...][None, :]
           + jnp.sum(xd_ref[...] * wfm_ref[...], axis=1, keepdims=True)
           + bfm_ref[...][None, :])
    fm2 = fm2_ref[:, :1]
    out_ref[...] = jax.nn.sigmoid(fm1 + fm2 + d)


def _tc2(out1, fm2, fg, xd, wfm, bfm, bias, g1, be1, w2t, b2, g2, be2, wo, bo):
    return pl.pallas_call(
        _tc2_body,
        out_shape=jax.ShapeDtypeStruct((B, 1), jnp.float32),
    )(out1, fm2, fg, xd, wfm, bfm, bias, g1, be1, w2t, b2, g2, be2, wo, bo)


def kernel(X_sparse, X_dense, fm1_emb, bias, w_fm1_dense, b_fm1_dense,
           emb_tables, w_dense, b_dense, w1, b1, g1, be1, w2, b2, g2, be2,
           w_out, b_out):
    pj = jnp.asarray(_PJ)
    colbase = jnp.asarray(_COLBASE)
    old = jnp.asarray(_OLD)
    idx = colbase[None, :] + X_sparse[:, pj] * NF
    idx3 = idx.reshape(NW, NCH, CH)
    offs = jnp.arange(NF, dtype=X_sparse.dtype) * VOCAB
    idxf = (X_sparse + offs[None, :]).reshape(NW, FNCH, FCH)

    emb_flat = jnp.transpose(emb_tables, (1, 0, 2)).reshape(NF * TOTAL, EMB)
    f16 = jnp.pad(fm1_emb, ((0, 0), (0, EMB - 1)))

    w1p = w1.reshape(256, NPOS, EMB)[:, old, :].reshape(256, DNN_IN)
    wdp = w_dense.reshape(NPOS, EMB, ND)[old].reshape(DNN_IN, ND)
    bdp = b_dense.reshape(NPOS, EMB)[old].reshape(DNN_IN)

    gflat, fflat = _sc_gather(emb_flat, idx3, f16, idxf)
    g = gflat.reshape(B, DNN_IN)
    fg = fflat.reshape(B, NF * EMB)

    out1, fm2 = _tc1(g, X_dense, wdp, bdp, w1p, b1)
    return _tc2(out1, fm2, fg, X_dense, w_fm1_dense, b_fm1_dense, bias,
                g1, be1, w2, b2, g2, be2, w_out, b_out)


# speedup vs baseline: 3.2415x; 3.2415x over previous
import functools

import numpy as np
import jax
import jax.numpy as jnp
from jax import lax
from jax.experimental import pallas as pl
from jax.experimental.pallas import tpu as pltpu
from jax.experimental.pallas import tpu_sc as plsc

NF = 26
VOCAB = 1000
TOTAL = NF * VOCAB
EMB = 16
B = 1024
ND = 13
NPOS = NF * NF
DNN_IN = NPOS * EMB
HALF = (NPOS // 2) * EMB
NPAIRS = (NF * (NF - 1)) // 2

_pairs = [(i, j) for i in range(NF) for j in range(i + 1, NF)]
_left = _pairs + [(d, d) for d in range(13)]
_right = [(j, i) for (i, j) in _pairs] + [(d, d) for d in range(13, NF)]
_order = _left + _right
_PI = np.array([p[0] for p in _order], dtype=np.int32)
_PJ = np.array([p[1] for p in _order], dtype=np.int32)
_OLD = _PI * NF + _PJ
_COLBASE = _PJ * VOCAB * NF + _PI

NW = 32
NROWS = B * NPOS
RPW = NROWS // NW
CH = 128
NCH = RPW // CH
GRP = 13
NGRP = NCH // GRP
FROWS = B * NF
FRPW = FROWS // NW
FCH = 64
FNCH = FRPW // FCH


def _sc_gather(emb_flat, idx3, f16, idxf3):
    mesh = plsc.VectorSubcoreMesh(core_axis_name="c", subcore_axis_name="s")
    nc = mesh.num_cores

    @functools.partial(
        pl.kernel,
        out_type=[
            jax.ShapeDtypeStruct((NROWS, EMB), jnp.float32),
            jax.ShapeDtypeStruct((FROWS, EMB), jnp.float32),
        ],
        mesh=mesh,
        compiler_params=pltpu.CompilerParams(use_tc_tiling_on_sc=False),
        scratch_types=(
            [pltpu.VMEM((NCH, CH), jnp.int32),
             pltpu.VMEM((FNCH, FCH), jnp.int32)]
            + [pltpu.VMEM((CH, EMB), jnp.float32) for _ in range(GRP)]
            + [pltpu.VMEM((FCH, EMB), jnp.float32) for _ in range(FNCH)]
            + [pltpu.SemaphoreType.DMA, pltpu.SemaphoreType.DMA]
        ),
    )
    def k(emb_hbm, idx_hbm, f_hbm, idxf_hbm, gout, fout, idx_v, idxf_v, *rest):
        bufs = rest[:GRP]
        fbufs = rest[GRP:GRP + FNCH]
        sem_g = rest[GRP + FNCH]
        sem_o = rest[GRP + FNCH + 1]
        wid = lax.axis_index("s") * nc + lax.axis_index("c")
        pltpu.sync_copy(idx_hbm.at[wid], idx_v)
        pltpu.sync_copy(idxf_hbm.at[wid], idxf_v)

        fbase = wid * FRPW
        fdescs = [pltpu.async_copy(f_hbm.at[idxf_v.at[c]], fbufs[c], sem_g)
                  for c in range(FNCH)]
        for d in fdescs:
            d.wait()
        odescs = [pltpu.async_copy(
            fbufs[c], fout.at[pl.ds(fbase + c * FCH, FCH)], sem_o)
            for c in range(FNCH)]
        for d in odescs:
            d.wait()

        base = wid * RPW

        def grp_body(g, carry):
            off = g * GRP
            descs = [pltpu.async_copy(
                emb_hbm.at[idx_v.at[off + c]], bufs[c], sem_g)
                for c in range(GRP)]
            for d in descs:
                d.wait()
            outs = [pltpu.async_copy(
                bufs[c], gout.at[pl.ds(base + (off + c) * CH, CH)], sem_o)
                for c in range(GRP)]
            for d in outs:
                d.wait()
            return carry

        lax.fori_loop(0, NGRP, grp_body, 0)

    return k(emb_flat, idx3, f16, idxf3)


def _tc0_body(t_ref, out_ref):
    out_ref[...] = jnp.swapaxes(t_ref[...], 0, 1)


def _tc0(tin):
    vch = 2048
    return pl.pallas_call(
        _tc0_body,
        grid=((TOTAL + vch - 1) // vch,),
        in_specs=[pl.BlockSpec((NF * EMB, vch), lambda k: (0, k))],
        out_specs=pl.BlockSpec((vch, NF * EMB), lambda k: (k, 0)),
        out_shape=jax.ShapeDtypeStruct((TOTAL, NF * EMB), jnp.float32),
    )(tin)


def _tc1_body(g_ref, xd_ref, wd_ref, bd_ref, w1_ref, b1_ref, out_ref, fm2_ref):
    gt = g_ref[...]
    dense = lax.dot_general(xd_ref[...], wd_ref[...],
                            (((1,), (1,)), ((), ())),
                            preferred_element_type=jnp.float32)
    dense = jnp.maximum(dense + bd_ref[...][None, :], 0.0)
    z = gt + dense
    out_ref[...] = (lax.dot_general(z, w1_ref[...],
                                    (((1,), (1,)), ((), ())),
                                    preferred_element_type=jnp.float32)
                    + b1_ref[...][None, :])
    lhs = gt[:, :HALF]
    rhs = gt[:, HALF:]
    mask = lax.broadcasted_iota(jnp.int32, (128, HALF), 1) < NPAIRS * EMB
    prod = jnp.where(mask, lhs * rhs, 0.0)
    fm2 = jnp.sum(prod, axis=1, keepdims=True)
    fm2_ref[...] = jnp.broadcast_to(fm2, (128, 128))


def _tc1(g, xd, wd, bd, w1, b1):
    return pl.pallas_call(
        _tc1_body,
        grid=(B // 128,),
        in_specs=[
            pl.BlockSpec((128, DNN_IN), lambda b: (b, 0)),
            pl.BlockSpec((128, ND), lambda b: (b, 0)),
            pl.BlockSpec((DNN_IN, ND), lambda b: (0, 0)),
            pl.BlockSpec((DNN_IN,), lambda b: (0,)),
            pl.BlockSpec((256, DNN_IN), lambda b: (0, 0)),
            pl.BlockSpec((256,), lambda b: (0,)),
        ],
        out_specs=[
            pl.BlockSpec((128, 256), lambda b: (b, 0)),
            pl.BlockSpec((128, 128), lambda b: (b, 0)),
        ],
        out_shape=[
            jax.ShapeDtypeStruct((B, 256), jnp.float32),
            jax.ShapeDtypeStruct((B, 128), jnp.float32),
        ],
    )(g, xd, wd, bd, w1, b1)


def _tc2_body(x_ref, fm2_ref, fg_ref, xd_ref, wfm_ref, bfm_ref, bias_ref,
              g1_ref, be1_ref, w2t_ref, b2_ref, g2_ref, be2_ref,
              wo_ref, bo_ref, out_ref):
    eps = 1e-5
    x = x_ref[...]
    m1 = jnp.mean(x, axis=0)
    v1 = jnp.mean(x * x, axis=0) - m1 * m1
    h1 = (x - m1[None, :]) * lax.rsqrt(v1[None, :] + eps)
    h1 = jnp.maximum(h1 * g1_ref[...][None, :] + be1_ref[...][None, :], 0.0)
    h2 = lax.dot_general(h1, w2t_ref[...], (((1,), (1,)), ((), ())),
                         preferred_element_type=jnp.float32)
    h2 = h2 + b2_ref[...][None, :]
    m2 = jnp.mean(h2, axis=0)
    v2 = jnp.mean(h2 * h2, axis=0) - m2 * m2
    h2 = (h2 - m2[None, :]) * lax.rsqrt(v2[None, :] + eps)
    h2 = jnp.maximum(h2 * g2_ref[...][None, :] + be2_ref[...][None, :], 0.0)
    d = jnp.sum(h2 * wo_ref[...], axis=1, keepdims=True) + bo_ref[...][None, :]
    fm1 = (jnp.sum(fg_ref[...], axis=1, keepdims=True)
           + bias_ref[...][None, :]
           + jnp.sum(xd_ref[...] * wfm_ref[...], axis=1, keepdims=True)
           + bfm_ref[...][None, :])
    fm2 = fm2_ref[:, :1]
    out_ref[...] = jax.nn.sigmoid(fm1 + fm2 + d)


def _tc2(out1, fm2, fg, xd, wfm, bfm, bias, g1, be1, w2t, b2, g2, be2, wo, bo):
    return pl.pallas_call(
        _tc2_body,
        out_shape=jax.ShapeDtypeStruct((B, 1), jnp.float32),
    )(out1, fm2, fg, xd, wfm, bfm, bias, g1, be1, w2t, b2, g2, be2, wo, bo)


def kernel(X_sparse, X_dense, fm1_emb, bias, w_fm1_dense, b_fm1_dense,
           emb_tables, w_dense, b_dense, w1, b1, g1, be1, w2, b2, g2, be2,
           w_out, b_out):
    pj = jnp.asarray(_PJ)
    colbase = jnp.asarray(_COLBASE)
    old = jnp.asarray(_OLD)
    idx = colbase[None, :] + X_sparse[:, pj] * NF
    idx3 = idx.reshape(NW, NCH, CH)
    offs = jnp.arange(NF, dtype=X_sparse.dtype) * VOCAB
    idxf = (X_sparse + offs[None, :]).reshape(NW, FNCH, FCH)

    tin = jnp.transpose(emb_tables, (0, 2, 1)).reshape(NF * EMB, TOTAL)
    emb_flat = _tc0(tin).reshape(NF * TOTAL, EMB)
    f16 = jnp.pad(fm1_emb, ((0, 0), (0, EMB - 1)))

    w1p = w1.reshape(256, NPOS, EMB)[:, old, :].reshape(256, DNN_IN)
    wdp = w_dense.reshape(NPOS, EMB, ND)[old].reshape(DNN_IN, ND)
    bdp = b_dense.reshape(NPOS, EMB)[old].reshape(DNN_IN)

    gflat, fflat = _sc_gather(emb_flat, idx3, f16, idxf)
    g = gflat.reshape(B, DNN_IN)
    fg = fflat.reshape(B, NF * EMB)

    out1, fm2 = _tc1(g, X_dense, wdp, bdp, w1p, b1)
    return _tc2(out1, fm2, fg, X_dense, w_fm1_dense, b_fm1_dense, bias,
                g1, be1, w2, b2, g2, be2, w_out, b_out)
